# baseline (speedup 1.0000x reference)
import functools

import jax
import jax.numpy as jnp
from jax import lax
from jax.experimental import pallas as pl
from jax.experimental.pallas import tpu as pltpu
from jax.experimental.pallas import tpu_sc as plsc

B, N, M, K, CIN, COUT = 4, 16384, 4096, 16, 32, 32
EPS = 1e-5
R = B * M * K
MK = M * K

NC, NS = 2, 16
NW = NC * NS
IPW = R // NW
WPB = NW // B
CHUNK = 128
NCHUNK = IPW // CHUNK
NSLOT = 8

PBLK = 2048
CBLK = PBLK // K
GRID = MK // PBLK

_ERF_C = (0.7976951671394331, -0.13235113448832125, 0.01933003906485979,
          -0.0020965061023684663, 0.00016135125785942924,
          -8.203207474327787e-06, 2.4348881578518944e-07,
          -3.162766243347548e-09)


def _gelu(x):
    xc = jnp.clip(x, -4.0, 4.0)
    t = xc * xc
    p = jnp.float32(_ERF_C[7])
    for k in range(6, -1, -1):
        p = p * t + jnp.float32(_ERF_C[k])
    erf = jnp.clip(p * xc, -1.0, 1.0)
    return 0.5 * x * (1.0 + erf)


def _fold4(v):
    t = v[:, 0:32] + v[:, 32:64] + v[:, 64:96] + v[:, 96:128]
    return jnp.concatenate([t, t, t, t], axis=1)


def _dotT(a, b):
    return lax.dot_general(a, b, (((0,), (0,)), ((), ())),
                           preferred_element_type=jnp.float32)


def _sums(x, out_ref):
    ones = jnp.ones((1, x.shape[0]), jnp.float32)
    out_ref[0:1, :] += jnp.dot(ones, x, preferred_element_type=jnp.float32)
    out_ref[1:2, :] += jnp.dot(ones, x * x,
                               preferred_element_type=jnp.float32)


def _proj_u_kernel(posT_ref, featT_ref, w1a_ref, w1b_ref, u_ref):
    cols = []
    for b in range(B):
        cols.append(_dotT(posT_ref[b], w1a_ref[...])
                    + _dotT(featT_ref[b], w1b_ref[...]))
    u_ref[...] = jnp.concatenate(cols, axis=1).astype(jnp.bfloat16)


def _proj_c_kernel(cposT_ref, cfT_ref, w1a_ref, eye_ref, b1_ref, c_ref,
                   cf_ref):
    ccols = []
    fcols = []
    for b in range(B):
        ccols.append(b1_ref[...] - _dotT(cposT_ref[b], w1a_ref[...]))
        fcols.append(_dotT(cfT_ref[b], eye_ref[...]))
    c_ref[...] = jnp.concatenate(ccols, axis=1)
    cf_ref[...] = jnp.concatenate(fcols, axis=1)


def _sc_gather(u_rows, idx_flat):
    mesh = plsc.VectorSubcoreMesh(core_axis_name="c", subcore_axis_name="s")

    @functools.partial(
        pl.kernel,
        out_type=jax.ShapeDtypeStruct((MK, B * COUT), jnp.bfloat16),
        mesh=mesh,
        scratch_types=(
            [pltpu.VMEM((IPW,), jnp.int32)]
            + [pltpu.VMEM((CHUNK, COUT), jnp.bfloat16) for _ in range(NSLOT)]
            + [pltpu.SemaphoreType.DMA for _ in range(2 * NSLOT)]
        ),
        compiler_params=pltpu.CompilerParams(use_tc_tiling_on_sc=False),
    )
    def k(u_hbm, idx_hbm, g_hbm, idx_v, *rest):
        bufs = rest[:NSLOT]
        gsem = rest[NSLOT:2 * NSLOT]
        wsem = rest[2 * NSLOT:]
        wid = lax.axis_index("s") * NC + lax.axis_index("c")
        base = wid * IPW
        b = wid // WPB
        lbase = base - b * MK
        lane0 = b * COUT
        pltpu.sync_copy(idx_hbm.at[pl.ds(base, IPW)], idx_v)

        def g_dst(j):
            return g_hbm.at[pl.ds(lbase + j * CHUNK, CHUNK),
                            pl.ds(lane0, COUT)]

        def fire(s, j):
            pltpu.async_copy(
                u_hbm.at[idx_v.at[pl.ds(j * CHUNK, CHUNK)]], bufs[s], gsem[s])

        def gwait(s):
            pltpu.make_async_copy(
                u_hbm.at[idx_v.at[pl.ds(0, CHUNK)]], bufs[s], gsem[s]).wait()

        def wfire(s, j):
            pltpu.async_copy(bufs[s], g_dst(j), wsem[s])

        def wwait(s):
            pltpu.make_async_copy(bufs[s], g_dst(0), wsem[s]).wait()

        for s in range(NSLOT):
            fire(s, s)

        def body(t, _):
            for s in range(NSLOT):
                gwait(s)
                wfire(s, NSLOT * t + s)
            for s in range(NSLOT):
                wwait(s)
                fire(s, NSLOT * (t + 1) + s)
            return 0

        lax.fori_loop(0, NCHUNK // NSLOT - 1, body, 0)

        for s in range(NSLOT):
            gwait(s)
            wfire(s, NCHUNK - NSLOT + s)
        for s in range(NSLOT):
            wwait(s)

    return k(u_rows, idx_flat)


def _c_packed(c_ref):
    c = c_ref[...]
    return jnp.broadcast_to(c[:, None, :], (CBLK, K, 128)).reshape(PBLK, 128)


def _bn_coefs(stats_ref, gamma, beta):
    s = _fold4(stats_ref[0:1, :])
    q = _fold4(stats_ref[1:2, :])
    mean = s * (1.0 / R)
    var = q * (1.0 / R) - mean * mean
    a = gamma * lax.rsqrt(var + EPS)
    c = beta - mean * a
    return a, c


def _stats1_kernel(g_ref, c_ref, out_ref):
    i = pl.program_id(0)

    @pl.when(i == 0)
    def _():
        out_ref[...] = jnp.zeros_like(out_ref)

    h = g_ref[...].astype(jnp.float32) + _c_packed(c_ref)
    _sums(h, out_ref)


def _stats2_kernel(g_ref, c_ref, stats1_ref, vecs_ref, w2bd_ref, y_ref,
                   out_ref):
    i = pl.program_id(0)

    @pl.when(i == 0)
    def _():
        out_ref[...] = jnp.zeros_like(out_ref)

    a1, c1 = _bn_coefs(stats1_ref, vecs_ref[1:2, :], vecs_ref[2:3, :])
    cc = a1 * c_ref[...] + c1
    ccr = jnp.broadcast_to(cc[:, None, :], (CBLK, K, 128)).reshape(PBLK, 128)
    u = _gelu(a1 * g_ref[...].astype(jnp.float32) + ccr)
    y = jnp.dot(u, w2bd_ref[...], preferred_element_type=jnp.float32)
    y = y + vecs_ref[3:4, :]
    y_ref[...] = y.astype(jnp.bfloat16)
    _sums(y, out_ref)


def _final_kernel(y_ref, stats2_ref, vecs_ref, cf_ref, out_ref):
    a2, c2 = _bn_coefs(stats2_ref, vecs_ref[4:5, :], vecs_ref[5:6, :])
    y = y_ref[...].reshape(CBLK, K, 128)
    ymx = jnp.max(y, axis=1).astype(jnp.float32)
    ymn = jnp.min(y, axis=1).astype(jnp.float32)
    va = a2 * ymx + c2
    vb = a2 * ymn + c2
    m1 = jnp.maximum(va, vb)
    m0 = jnp.minimum(va, vb)
    out_ref[...] = jnp.maximum(_gelu(m1), _gelu(m0)) + cf_ref[...]


def _block_diag4(w):
    bd = jnp.zeros((128, 128), jnp.float32)
    for t in range(4):
        bd = bd.at[t * 32:(t + 1) * 32, t * 32:(t + 1) * 32].set(w)
    return bd


def _tile4(v):
    return jnp.tile(v.reshape(1, COUT), (1, 4)).reshape(128)


def kernel(feat, pos, center_feat, center_pos, knn_idx, W1, b1, g1, be1,
           W2, b2, g2, be2):
    f32 = jnp.float32
    w1a3 = W1[:, :3].T
    w2bd = _block_diag4(W2.T)
    b1r = b1.reshape(1, COUT)
    eye = jnp.eye(COUT, dtype=f32)
    vecs = jnp.zeros((8, 128), f32)
    vecs = vecs.at[1, :].set(_tile4(g1)).at[2, :].set(_tile4(be1))
    vecs = vecs.at[3, :].set(_tile4(b2))
    vecs = vecs.at[4, :].set(_tile4(g2)).at[5, :].set(_tile4(be2))

    posT = pos.transpose(0, 2, 1)
    featT = feat.transpose(0, 2, 1)
    cposT = center_pos.transpose(0, 2, 1)
    cfT = center_feat.transpose(0, 2, 1)

    idx_flat = (knn_idx * 4 + jnp.arange(B, dtype=jnp.int32)[:, None, None]
                ).reshape(R)

    ublk = 2048
    u_packed = pl.pallas_call(
        _proj_u_kernel,
        grid=(N // ublk,),
        in_specs=[
            pl.BlockSpec((B, 3, ublk), lambda j: (0, 0, j)),
            pl.BlockSpec((B, CIN, ublk), lambda j: (0, 0, j)),
            pl.BlockSpec((3, COUT), lambda j: (0, 0)),
            pl.BlockSpec((CIN, COUT), lambda j: (0, 0)),
        ],
        out_specs=pl.BlockSpec((ublk, B * COUT), lambda j: (j, 0)),
        out_shape=jax.ShapeDtypeStruct((N, B * COUT), jnp.bfloat16),
    )(posT, featT, w1a3, W1[:, 3:].T)

    c_tab, cf_packed = pl.pallas_call(
        _proj_c_kernel,
        grid=(M // ublk,),
        in_specs=[
            pl.BlockSpec((B, 3, ublk), lambda j: (0, 0, j)),
            pl.BlockSpec((B, COUT, ublk), lambda j: (0, 0, j)),
            pl.BlockSpec((3, COUT), lambda j: (0, 0)),
            pl.BlockSpec((COUT, COUT), lambda j: (0, 0)),
            pl.BlockSpec((1, COUT), lambda j: (0, 0)),
        ],
        out_specs=[pl.BlockSpec((ublk, B * COUT), lambda j: (j, 0)),
                   pl.BlockSpec((ublk, B * COUT), lambda j: (j, 0))],
        out_shape=[jax.ShapeDtypeStruct((M, B * COUT), f32),
                   jax.ShapeDtypeStruct((M, B * COUT), f32)],
    )(cposT, cfT, w1a3, eye, b1r)

    g_packed = _sc_gather(u_packed.reshape(B * N, COUT), idx_flat)

    g_spec = pl.BlockSpec((PBLK, 128), lambda i: (i, 0))
    c_spec = pl.BlockSpec((CBLK, 128), lambda i: (i, 0))
    full8 = pl.BlockSpec((8, 128), lambda i: (0, 0))
    w_spec = pl.BlockSpec((128, 128), lambda i: (0, 0))
    acc_spec = pl.BlockSpec((8, 128), lambda i: (0, 0))

    stats1 = pl.pallas_call(
        _stats1_kernel,
        grid=(GRID,),
        in_specs=[g_spec, c_spec],
        out_specs=acc_spec,
        out_shape=jax.ShapeDtypeStruct((8, 128), f32),
    )(g_packed, c_tab)

    y_packed, stats2 = pl.pallas_call(
        _stats2_kernel,
        grid=(GRID,),
        in_specs=[g_spec, c_spec, full8, full8, w_spec],
        out_specs=[g_spec, acc_spec],
        out_shape=[jax.ShapeDtypeStruct((MK, 128), jnp.bfloat16),
                   jax.ShapeDtypeStruct((8, 128), f32)],
    )(g_packed, c_tab, stats1, vecs, w2bd)

    out_packed = pl.pallas_call(
        _final_kernel,
        grid=(GRID,),
        in_specs=[g_spec, full8, full8, c_spec],
        out_specs=pl.BlockSpec((CBLK, 128), lambda i: (i, 0)),
        out_shape=jax.ShapeDtypeStruct((M, B * COUT), f32),
    )(y_packed, stats2, vecs, cf_packed)

    return out_packed.reshape(M, B, COUT).transpose(1, 0, 2)


# speedup vs baseline: 1.3836x; 1.3836x over previous
import functools

import jax
import jax.numpy as jnp
from jax import lax
from jax.experimental import pallas as pl
from jax.experimental.pallas import tpu as pltpu
from jax.experimental.pallas import tpu_sc as plsc

B, N, M, K, CIN, COUT = 4, 16384, 4096, 16, 32, 32
EPS = 1e-5
R = B * M * K
MK = M * K

NC, NS = 2, 16
NW = NC * NS
IPW = R // NW
WPB = NW // B
CHUNK = 128
NCHUNK = IPW // CHUNK
NSLOT = 8

PBLK = 2048
CBLK = PBLK // K
GRID = MK // PBLK

_ERF_C = (0.7976951671394331, -0.13235113448832125, 0.01933003906485979,
          -0.0020965061023684663, 0.00016135125785942924,
          -8.203207474327787e-06, 2.4348881578518944e-07,
          -3.162766243347548e-09)


def _gelu(x):
    xc = jnp.clip(x, -4.0, 4.0)
    t = xc * xc
    p = jnp.float32(_ERF_C[7])
    for k in range(6, -1, -1):
        p = p * t + jnp.float32(_ERF_C[k])
    erf = jnp.clip(p * xc, -1.0, 1.0)
    return 0.5 * x * (1.0 + erf)


def _fold4(v):
    t = v[:, 0:32] + v[:, 32:64] + v[:, 64:96] + v[:, 96:128]
    return jnp.concatenate([t, t, t, t], axis=1)


def _dotT(a, b):
    return lax.dot_general(a, b, (((0,), (0,)), ((), ())),
                           preferred_element_type=jnp.float32)


def _sums(x, out_ref):
    ones = jnp.ones((1, x.shape[0]), jnp.float32)
    out_ref[0:1, :] += jnp.dot(ones, x, preferred_element_type=jnp.float32)
    out_ref[1:2, :] += jnp.dot(ones, x * x,
                               preferred_element_type=jnp.float32)


def _proj_u_kernel(posT_ref, featT_ref, w1a_ref, w1b_ref, u_ref):
    cols = []
    for b in range(B):
        cols.append(_dotT(posT_ref[b], w1a_ref[...])
                    + _dotT(featT_ref[b], w1b_ref[...]))
    u_ref[...] = jnp.concatenate(cols, axis=1)


def _proj_c_kernel(cposT_ref, cfT_ref, w1a_ref, eye_ref, b1_ref, c_ref,
                   cf_ref):
    ccols = []
    fcols = []
    for b in range(B):
        ccols.append(b1_ref[...] - _dotT(cposT_ref[b], w1a_ref[...]))
        fcols.append(_dotT(cfT_ref[b], eye_ref[...]))
    c_ref[...] = jnp.concatenate(ccols, axis=1)
    cf_ref[...] = jnp.concatenate(fcols, axis=1)


def _sc_gather(u_rows, idx_flat):
    mesh = plsc.VectorSubcoreMesh(core_axis_name="c", subcore_axis_name="s")

    @functools.partial(
        pl.kernel,
        out_type=jax.ShapeDtypeStruct((MK, B * COUT), jnp.float32),
        mesh=mesh,
        scratch_types=(
            [pltpu.VMEM((IPW,), jnp.int32)]
            + [pltpu.VMEM((CHUNK, COUT), jnp.float32) for _ in range(NSLOT)]
            + [pltpu.SemaphoreType.DMA for _ in range(2 * NSLOT)]
        ),
        compiler_params=pltpu.CompilerParams(use_tc_tiling_on_sc=False),
    )
    def k(u_hbm, idx_hbm, g_hbm, idx_v, *rest):
        bufs = rest[:NSLOT]
        gsem = rest[NSLOT:2 * NSLOT]
        wsem = rest[2 * NSLOT:]
        wid = lax.axis_index("s") * NC + lax.axis_index("c")
        base = wid * IPW
        b = wid // WPB
        lbase = base - b * MK
        lane0 = b * COUT
        pltpu.sync_copy(idx_hbm.at[pl.ds(base, IPW)], idx_v)

        def g_dst(j):
            return g_hbm.at[pl.ds(lbase + j * CHUNK, CHUNK),
                            pl.ds(lane0, COUT)]

        def fire(s, j):
            pltpu.async_copy(
                u_hbm.at[idx_v.at[pl.ds(j * CHUNK, CHUNK)]], bufs[s], gsem[s])

        def gwait(s):
            pltpu.make_async_copy(
                u_hbm.at[idx_v.at[pl.ds(0, CHUNK)]], bufs[s], gsem[s]).wait()

        def wfire(s, j):
            pltpu.async_copy(bufs[s], g_dst(j), wsem[s])

        def wwait(s):
            pltpu.make_async_copy(bufs[s], g_dst(0), wsem[s]).wait()

        for s in range(NSLOT):
            fire(s, s)

        def body(t, _):
            for s in range(NSLOT):
                gwait(s)
                wfire(s, NSLOT * t + s)
            for s in range(NSLOT):
                wwait(s)
                fire(s, NSLOT * (t + 1) + s)
            return 0

        lax.fori_loop(0, NCHUNK // NSLOT - 1, body, 0)

        for s in range(NSLOT):
            gwait(s)
            wfire(s, NCHUNK - NSLOT + s)
        for s in range(NSLOT):
            wwait(s)

    return k(u_rows, idx_flat)


def _c_packed(c_ref):
    c = c_ref[...]
    return jnp.broadcast_to(c[:, None, :], (CBLK, K, 128)).reshape(PBLK, 128)


def _bn_coefs(stats_ref, gamma, beta):
    s = _fold4(stats_ref[0:1, :])
    q = _fold4(stats_ref[1:2, :])
    mean = s * (1.0 / R)
    var = q * (1.0 / R) - mean * mean
    a = gamma * lax.rsqrt(var + EPS)
    c = beta - mean * a
    return a, c


def _stats1_kernel(g_ref, c_ref, out_ref):
    i = pl.program_id(0)

    @pl.when(i == 0)
    def _():
        out_ref[...] = jnp.zeros_like(out_ref)

    h = g_ref[...] + _c_packed(c_ref)
    _sums(h, out_ref)


def _stats2_kernel(g_ref, c_ref, stats1_ref, vecs_ref, w2bd_ref, y_ref,
                   out_ref):
    i = pl.program_id(0)

    @pl.when(i == 0)
    def _():
        out_ref[...] = jnp.zeros_like(out_ref)

    a1, c1 = _bn_coefs(stats1_ref, vecs_ref[1:2, :], vecs_ref[2:3, :])
    cc = a1 * c_ref[...] + c1
    ccr = jnp.broadcast_to(cc[:, None, :], (CBLK, K, 128)).reshape(PBLK, 128)
    u = _gelu(a1 * g_ref[...] + ccr)
    y = jnp.dot(u, w2bd_ref[...], preferred_element_type=jnp.float32)
    y = y + vecs_ref[3:4, :]
    y_ref[...] = y
    _sums(y, out_ref)


def _final_kernel(y_ref, stats2_ref, vecs_ref, cf_ref, out_ref):
    a2, c2 = _bn_coefs(stats2_ref, vecs_ref[4:5, :], vecs_ref[5:6, :])
    y = y_ref[...].reshape(CBLK, K, 128)
    ymx = jnp.max(y, axis=1)
    ymn = jnp.min(y, axis=1)
    va = a2 * ymx + c2
    vb = a2 * ymn + c2
    m1 = jnp.maximum(va, vb)
    m0 = jnp.minimum(va, vb)
    out_ref[...] = jnp.maximum(_gelu(m1), _gelu(m0)) + cf_ref[...]


def _block_diag4(w):
    bd = jnp.zeros((128, 128), jnp.float32)
    for t in range(4):
        bd = bd.at[t * 32:(t + 1) * 32, t * 32:(t + 1) * 32].set(w)
    return bd


def _tile4(v):
    return jnp.tile(v.reshape(1, COUT), (1, 4)).reshape(128)


def kernel(feat, pos, center_feat, center_pos, knn_idx, W1, b1, g1, be1,
           W2, b2, g2, be2):
    f32 = jnp.float32
    w1a3 = W1[:, :3].T
    w2bd = _block_diag4(W2.T)
    b1r = b1.reshape(1, COUT)
    eye = jnp.eye(COUT, dtype=f32)
    vecs = jnp.zeros((8, 128), f32)
    vecs = vecs.at[1, :].set(_tile4(g1)).at[2, :].set(_tile4(be1))
    vecs = vecs.at[3, :].set(_tile4(b2))
    vecs = vecs.at[4, :].set(_tile4(g2)).at[5, :].set(_tile4(be2))

    posT = pos.transpose(0, 2, 1)
    featT = feat.transpose(0, 2, 1)
    cposT = center_pos.transpose(0, 2, 1)
    cfT = center_feat.transpose(0, 2, 1)

    idx_flat = (knn_idx * 4 + jnp.arange(B, dtype=jnp.int32)[:, None, None]
                ).reshape(R)

    ublk = 2048
    u_packed = pl.pallas_call(
        _proj_u_kernel,
        grid=(N // ublk,),
        in_specs=[
            pl.BlockSpec((B, 3, ublk), lambda j: (0, 0, j)),
            pl.BlockSpec((B, CIN, ublk), lambda j: (0, 0, j)),
            pl.BlockSpec((3, COUT), lambda j: (0, 0)),
            pl.BlockSpec((CIN, COUT), lambda j: (0, 0)),
        ],
        out_specs=pl.BlockSpec((ublk, B * COUT), lambda j: (j, 0)),
        out_shape=jax.ShapeDtypeStruct((N, B * COUT), f32),
    )(posT, featT, w1a3, W1[:, 3:].T)

    c_tab, cf_packed = pl.pallas_call(
        _proj_c_kernel,
        grid=(M // ublk,),
        in_specs=[
            pl.BlockSpec((B, 3, ublk), lambda j: (0, 0, j)),
            pl.BlockSpec((B, COUT, ublk), lambda j: (0, 0, j)),
            pl.BlockSpec((3, COUT), lambda j: (0, 0)),
            pl.BlockSpec((COUT, COUT), lambda j: (0, 0)),
            pl.BlockSpec((1, COUT), lambda j: (0, 0)),
        ],
        out_specs=[pl.BlockSpec((ublk, B * COUT), lambda j: (j, 0)),
                   pl.BlockSpec((ublk, B * COUT), lambda j: (j, 0))],
        out_shape=[jax.ShapeDtypeStruct((M, B * COUT), f32),
                   jax.ShapeDtypeStruct((M, B * COUT), f32)],
    )(cposT, cfT, w1a3, eye, b1r)

    g_packed = _sc_gather(u_packed.reshape(B * N, COUT), idx_flat)

    g_spec = pl.BlockSpec((PBLK, 128), lambda i: (i, 0))
    c_spec = pl.BlockSpec((CBLK, 128), lambda i: (i, 0))
    full8 = pl.BlockSpec((8, 128), lambda i: (0, 0))
    w_spec = pl.BlockSpec((128, 128), lambda i: (0, 0))
    acc_spec = pl.BlockSpec((8, 128), lambda i: (0, 0))

    stats1 = pl.pallas_call(
        _stats1_kernel,
        grid=(GRID,),
        in_specs=[g_spec, c_spec],
        out_specs=acc_spec,
        out_shape=jax.ShapeDtypeStruct((8, 128), f32),
    )(g_packed, c_tab)

    y_packed, stats2 = pl.pallas_call(
        _stats2_kernel,
        grid=(GRID,),
        in_specs=[g_spec, c_spec, full8, full8, w_spec],
        out_specs=[g_spec, acc_spec],
        out_shape=[jax.ShapeDtypeStruct((MK, 128), f32),
                   jax.ShapeDtypeStruct((8, 128), f32)],
    )(g_packed, c_tab, stats1, vecs, w2bd)

    out_packed = pl.pallas_call(
        _final_kernel,
        grid=(GRID,),
        in_specs=[g_spec, full8, full8, c_spec],
        out_specs=pl.BlockSpec((CBLK, 128), lambda i: (i, 0)),
        out_shape=jax.ShapeDtypeStruct((M, B * COUT), f32),
    )(y_packed, stats2, vecs, cf_packed)

    return out_packed.reshape(M, B, COUT).transpose(1, 0, 2)
